# baseline (speedup 1.0000x reference)
import functools

import jax
import jax.numpy as jnp
from jax import lax
from jax.experimental import pallas as pl
from jax.experimental.pallas import tpu as pltpu
from jax.experimental.pallas import tpu_sc as plsc

B = 16384
N_CAT = 26
N_NUM = 8
VOCAB = 100000
CHANNELS = 16
N_COLS = N_CAT + N_NUM
INV = 1.0 / N_COLS

NC = 2
NS = 16
NW = NC * NS
ROWS_PER_W = B // NW

SLAB = 12504
UNIT_V = 1024
MAIN_UNITS = VOCAB // UNIT_V
TAIL_V0 = VOCAB - 992

CHUNK = 128
CHUNKS_PER_W = ROWS_PER_W // CHUNK
LOOK = CHUNK * N_CAT
SUB = 256
N_SUB = LOOK // SUB


def _fin_body(part_ref, num_ref, w_ref, b_ref, out_ref):
    b_sum = jnp.sum(b_ref[...], axis=0, keepdims=True)
    out_ref[...] = part_ref[...] + (
        jnp.dot(num_ref[...], w_ref[...], preferred_element_type=jnp.float32)
        + b_sum
    ) * INV


def _finalize(partial, num_feat, lin_w, lin_b):
    return pl.pallas_call(
        _fin_body,
        out_shape=jax.ShapeDtypeStruct((B, CHANNELS), jnp.float32),
    )(partial, num_feat, lin_w, lin_b)


@functools.partial(
    pl.kernel,
    out_type=jax.ShapeDtypeStruct((N_CAT * SLAB, 128), jnp.float32),
    mesh=plsc.VectorSubcoreMesh(core_axis_name="c", subcore_axis_name="s"),
    compiler_params=pltpu.CompilerParams(
        use_tc_tiling_on_sc=True, needs_layout_passes=False
    ),
    scratch_types=[
        pltpu.VMEM((CHANNELS, UNIT_V), jnp.float32),
        pltpu.VMEM((UNIT_V // 8, 128), jnp.float32),
    ],
)
def _sc_detile(emb_tr, emb_tail, tout, in_v, blk_v):
    wid = lax.axis_index("s") * NC + lax.axis_index("c")
    lanes = lax.iota(jnp.int32, 16)

    def transpose_block():
        def pr_body(pr, carry):
            for k in range(8):
                col = plsc.load_gather(in_v, [lanes, pr * 8 + k + lanes * 0])
                blk_v[pr, pl.ds(k * CHANNELS, CHANNELS)] = col
            return carry

        lax.fori_loop(0, UNIT_V // 8, pr_body, 0)

    n_main = N_CAT * MAIN_UNITS

    def unit_body(i, carry):
        u = wid + i * NW
        j = u // MAIN_UNITS
        ub = u % MAIN_UNITS
        pltpu.sync_copy(emb_tr.at[j, :, pl.ds(ub * UNIT_V, UNIT_V)], in_v)
        transpose_block()
        pltpu.sync_copy(
            blk_v, tout.at[pl.ds(j * SLAB + ub * (UNIT_V // 8), UNIT_V // 8)]
        )
        return carry

    n_mine = (n_main - wid + NW - 1) // NW
    lax.fori_loop(0, n_mine, unit_body, 0)

    @pl.when(wid < N_CAT)
    def _tail():
        j = wid
        pltpu.sync_copy(emb_tail.at[j], in_v)
        transpose_block()
        pltpu.sync_copy(
            blk_v, tout.at[pl.ds(j * SLAB + TAIL_V0 // 8, UNIT_V // 8)]
        )


@functools.partial(
    pl.kernel,
    out_type=jax.ShapeDtypeStruct((B, CHANNELS), jnp.float32),
    mesh=plsc.VectorSubcoreMesh(core_axis_name="c", subcore_axis_name="s"),
    compiler_params=pltpu.CompilerParams(
        use_tc_tiling_on_sc=True, needs_layout_passes=False
    ),
    scratch_types=[
        pltpu.VMEM((N_CAT, ROWS_PER_W), jnp.int32),
        pltpu.VMEM((LOOK,), jnp.int32),
        pltpu.VMEM((LOOK,), jnp.int32),
        pltpu.VMEM((SUB, 128), jnp.float32),
        pltpu.VMEM((SUB, 128), jnp.float32),
        pltpu.VMEM((CHANNELS, CHUNK), jnp.float32),
        pltpu.VMEM((CHUNK, CHANNELS), jnp.float32),
        pltpu.SemaphoreType.DMA,
        pltpu.SemaphoreType.DMA,
    ],
)
def _sc_gather(table_p, idx_t, out, idx_v, pid_v, s16_v, rows0, rows1, acc_v,
               out_v, sem0, sem1):
    wid = lax.axis_index("s") * NC + lax.axis_index("c")
    base = wid * ROWS_PER_W
    lanes = lax.iota(jnp.int32, 16)
    pltpu.sync_copy(idx_t.at[:, pl.ds(base, ROWS_PER_W)], idx_v)
    bufs = ((rows0, sem0), (rows1, sem1))

    def fire(c, sb, buf, sem):
        for st in range(2):
            pltpu.async_copy(
                table_p.at[pid_v.at[pl.ds((sb * 2 + st) * 128, 128)]],
                buf.at[pl.ds(st * 128, 128)],
                sem,
            )

    def drain(buf, sem):
        pltpu.make_async_copy(table_p.at[pl.ds(0, SUB)], buf, sem).wait()

    def chunk_body(c, carry):
        row0 = base + c * CHUNK

        def pid_body(k, carry2):
            j = k // 8
            g = k - j * 8
            v16 = idx_v[j, pl.ds(c * CHUNK + g * 16, 16)]
            pid_v[pl.ds(k * 16, 16)] = (v16 >> 3) + j * SLAB
            s16_v[pl.ds(k * 16, 16)] = (v16 & 7) * CHANNELS
            return carry2

        lax.fori_loop(0, N_CAT * 8, pid_body, 0)

        def accum(sb, buf):
            def acc_body(hg, carry2):
                s16 = s16_v[pl.ds(sb * SUB + hg * 16, 16)]
                rid = hg * 16 + lanes
                sl = pl.ds((hg - (hg // 8) * 8) * 16, 16)
                for ch in range(CHANNELS):
                    val = plsc.load_gather(buf, [rid, s16 + ch])
                    acc_v[ch, sl] = acc_v[ch, sl] + val
                return carry2

            lax.fori_loop(0, 16, acc_body, 0)

        zero = jnp.zeros((16,), jnp.float32)

        def zero_body(g, carry2):
            for ch in range(CHANNELS):
                acc_v[ch, pl.ds(g * 16, 16)] = zero
            return carry2

        lax.fori_loop(0, 8, zero_body, 0)

        fire(c, 0, *bufs[0])
        for sb in range(N_SUB):
            buf, sem = bufs[sb % 2]
            drain(buf, sem)
            if sb + 1 < N_SUB:
                fire(c, sb + 1, *bufs[(sb + 1) % 2])
            accum(sb, buf)

        def tr_body(r, carry2):
            col = plsc.load_gather(acc_v, [lanes, r + lanes * 0])
            out_v[r] = col * INV
            return carry2

        lax.fori_loop(0, CHUNK, tr_body, 0)
        pltpu.sync_copy(out_v, out.at[pl.ds(row0, CHUNK)])
        return carry

    lax.fori_loop(0, CHUNKS_PER_W, chunk_body, 0)


def kernel(cat_idx, num_feat, emb_tables, lin_w, lin_b):
    emb_tr = jnp.swapaxes(emb_tables, 1, 2)
    emb_tail = jnp.pad(
        lax.slice(emb_tr, (0, 0, TAIL_V0), (N_CAT, CHANNELS, VOCAB)),
        ((0, 0), (0, 0), (0, UNIT_V - (VOCAB - TAIL_V0))),
    )
    idx_t = jnp.swapaxes(cat_idx.astype(jnp.int32), 0, 1)
    table_p = _sc_detile(emb_tr, emb_tail)
    partial = _sc_gather(table_p, idx_t)
    return _finalize(partial, num_feat, lin_w, lin_b)
